# Initial kernel scaffold; baseline (speedup 1.0000x reference)
import jax
import jax.numpy as jnp
from jax.experimental import pallas as pl


def kernel(entity, attribute, value_binned, time, W_entity, W_attribute, W_value_binned):
    raise NotImplementedError("write your pallas kernel here")


# speedup vs baseline: 2.7722x; 2.7722x over previous
import math

import jax
import jax.numpy as jnp
import numpy as np
from jax.experimental import pallas as pl
from jax.experimental.pallas import tpu as pltpu

_T = 128


def _tc_body(e_ref, a_ref, v_ref, t_ref, w_ref, qt_ref, rt_ref, o_ref):
    n_e = 32
    n_a = 16
    half = o_ref.shape[1] // 2

    e = e_ref[0, 0, :][None, :]
    a = a_ref[0, 0, :][None, :]
    v = v_ref[0, 0, :][None, :]
    t = t_ref[0, 0, :][None, :]

    ii = jax.lax.broadcasted_iota(jnp.int32, (w_ref.shape[0], _T), 0)
    oh_w = ((ii == e) | (ii == a + n_e) | (ii == v + n_e + n_a)).astype(jnp.float32)
    tok = jax.lax.dot_general(
        oh_w, w_ref[...], (((0,), (0,)), ((), ())),
        preferred_element_type=jnp.float32)

    q = jax.lax.shift_right_logical(t, 6)
    r = jax.lax.bitwise_and(t, 63)
    jj = jax.lax.broadcasted_iota(jnp.int32, (64, _T), 0)
    oh_q = (jj == q).astype(jnp.float32)
    oh_r = (jj == r).astype(jnp.float32)
    qrow = jax.lax.dot_general(
        oh_q, qt_ref[...], (((0,), (0,)), ((), ())),
        preferred_element_type=jnp.float32)
    rrow = jax.lax.dot_general(
        oh_r, rt_ref[...], (((0,), (0,)), ((), ())),
        preferred_element_type=jnp.float32)

    s1 = qrow[:, :half]
    c1 = qrow[:, half:]
    s2 = rrow[:, :half]
    c2 = rrow[:, half:]
    t_sin = s1 * c2 + c1 * s2
    t_cos = c1 * c2 - s1 * s2
    o_ref[...] = tok + jnp.concatenate([t_sin, t_cos], axis=1)


def kernel(entity, attribute, value_binned, time, W_entity, W_attribute, W_value_binned):
    B, S = entity.shape
    D = W_entity.shape[1]
    half = D // 2
    N = B * S
    G = N // _T

    ratio = math.log(10000.0) / half
    f = np.exp(-ratio * np.arange(half, dtype=np.float64))
    qa = (64.0 * np.arange(64, dtype=np.float64))[:, None] * f[None, :]
    ra = np.arange(64, dtype=np.float64)[:, None] * f[None, :]
    qtab = jnp.asarray(
        np.concatenate([np.sin(qa), np.cos(qa)], axis=1), dtype=jnp.float32)
    rtab = jnp.asarray(
        np.concatenate([np.sin(ra), np.cos(ra)], axis=1), dtype=jnp.float32)

    w_all = jnp.concatenate([W_entity, W_attribute, W_value_binned], axis=0)

    def r3(x):
        return x.reshape(G, 1, _T)

    idx_spec = pl.BlockSpec((1, 1, _T), lambda i: (i, 0, 0))
    tab_spec = lambda n: pl.BlockSpec((n, D), lambda i: (0, 0))

    out = pl.pallas_call(
        _tc_body,
        grid=(G,),
        in_specs=[idx_spec, idx_spec, idx_spec, idx_spec,
                  tab_spec(w_all.shape[0]), tab_spec(64), tab_spec(64)],
        out_specs=pl.BlockSpec((_T, D), lambda i: (i, 0)),
        out_shape=jax.ShapeDtypeStruct((N, D), jnp.float32),
        compiler_params=pltpu.CompilerParams(
            dimension_semantics=("arbitrary",)),
    )(r3(entity), r3(attribute), r3(value_binned), r3(time), w_all, qtab, rtab)
    return out.reshape(B, S, D)


# speedup vs baseline: 15.2375x; 5.4966x over previous
import math

import jax
import jax.numpy as jnp
import numpy as np
from jax.experimental import pallas as pl
from jax.experimental.pallas import tpu as pltpu

_L = 128
_TILES = 8
_T = _L * _TILES


def _tc_body(e_ref, a_ref, v_ref, t_ref, w_ref, qt_ref, rt_ref, o_ref):
    n_e = 32
    n_a = 16
    d = o_ref.shape[1]

    e2 = e_ref[0]
    a2 = a_ref[0]
    v2 = v_ref[0]
    t2 = t_ref[0]

    ii = jax.lax.broadcasted_iota(jnp.int32, (w_ref.shape[0], _L), 0)
    jj = jax.lax.broadcasted_iota(jnp.int32, (64, _L), 0)

    for k in range(_TILES):
        e = e2[k:k + 1, :]
        a = a2[k:k + 1, :]
        v = v2[k:k + 1, :]
        t = t2[k:k + 1, :]

        oh_w = ((ii == e) | (ii == a + n_e) | (ii == v + n_e + n_a)
                ).astype(jnp.float32)
        tok = jax.lax.dot_general(
            oh_w, w_ref[...], (((0,), (0,)), ((), ())),
            preferred_element_type=jnp.float32)

        q = jax.lax.shift_right_logical(t, 6)
        r = jax.lax.bitwise_and(t, 63)
        oh_q = (jj == q).astype(jnp.float32)
        oh_r = (jj == r).astype(jnp.float32)
        qq = jax.lax.dot_general(
            oh_q, qt_ref[...], (((0,), (0,)), ((), ())),
            preferred_element_type=jnp.float32)
        rr = jax.lax.dot_general(
            oh_r, rt_ref[...], (((0,), (0,)), ((), ())),
            preferred_element_type=jnp.float32)

        o_ref[k * _L:(k + 1) * _L, :] = (
            tok + qq[:, :d] * rr[:, :d] + qq[:, d:] * rr[:, d:])


def kernel(entity, attribute, value_binned, time, W_entity, W_attribute, W_value_binned):
    B, S = entity.shape
    D = W_entity.shape[1]
    half = D // 2
    N = B * S
    G = N // _T

    ratio = math.log(10000.0) / half
    f = np.exp(-ratio * np.arange(half, dtype=np.float64))
    qa = (64.0 * np.arange(64, dtype=np.float64))[:, None] * f[None, :]
    ra = np.arange(64, dtype=np.float64)[:, None] * f[None, :]
    s1, c1 = np.sin(qa), np.cos(qa)
    s2, c2 = np.sin(ra), np.cos(ra)
    qtab = jnp.asarray(np.concatenate([s1, c1, c1, s1], axis=1),
                       dtype=jnp.float32)
    rtab = jnp.asarray(np.concatenate([c2, c2, s2, -s2], axis=1),
                       dtype=jnp.float32)

    w_all = jnp.concatenate([W_entity, W_attribute, W_value_binned], axis=0)

    def r3(x):
        return x.reshape(G, _TILES, _L)

    idx_spec = pl.BlockSpec((1, _TILES, _L), lambda i: (i, 0, 0))

    out = pl.pallas_call(
        _tc_body,
        grid=(G,),
        in_specs=[idx_spec, idx_spec, idx_spec, idx_spec,
                  pl.BlockSpec((w_all.shape[0], D), lambda i: (0, 0)),
                  pl.BlockSpec((64, 2 * D), lambda i: (0, 0)),
                  pl.BlockSpec((64, 2 * D), lambda i: (0, 0))],
        out_specs=pl.BlockSpec((_T, D), lambda i: (i, 0)),
        out_shape=jax.ShapeDtypeStruct((N, D), jnp.float32),
        compiler_params=pltpu.CompilerParams(
            dimension_semantics=("arbitrary",)),
    )(r3(entity), r3(attribute), r3(value_binned), r3(time), w_all, qtab, rtab)
    return out.reshape(B, S, D)


# speedup vs baseline: 22.0920x; 1.4498x over previous
import math

import jax
import jax.numpy as jnp
import numpy as np
from jax.experimental import pallas as pl
from jax.experimental.pallas import tpu as pltpu

_L = 128
_TILES = 16
_T = _L * _TILES


def _tc_body(e_ref, a_ref, v_ref, t_ref, w_ref, qt_ref, rt_ref, o_ref):
    n_e = 32
    n_a = 16
    d = o_ref.shape[1]

    e2 = e_ref[0]
    a2 = a_ref[0]
    v2 = v_ref[0]
    t2 = t_ref[0]

    ii = jax.lax.broadcasted_iota(jnp.int32, (w_ref.shape[0], _L), 0)
    jj = jax.lax.broadcasted_iota(jnp.int32, (64, _L), 0)

    for k in range(_TILES):
        e = e2[k:k + 1, :]
        a = a2[k:k + 1, :]
        v = v2[k:k + 1, :]
        t = t2[k:k + 1, :]

        oh_w = ((ii == e) | (ii == a + n_e) | (ii == v + n_e + n_a)
                ).astype(jnp.bfloat16)
        tok = jax.lax.dot_general(
            oh_w, w_ref[...], (((0,), (0,)), ((), ())),
            preferred_element_type=jnp.float32)

        q = jax.lax.shift_right_logical(t, 6)
        r = jax.lax.bitwise_and(t, 63)
        oh_q = (jj == q).astype(jnp.bfloat16)
        oh_r = (jj == r).astype(jnp.bfloat16)
        qq = jax.lax.dot_general(
            oh_q, qt_ref[...], (((0,), (0,)), ((), ())),
            preferred_element_type=jnp.float32)
        rr = jax.lax.dot_general(
            oh_r, rt_ref[...], (((0,), (0,)), ((), ())),
            preferred_element_type=jnp.float32)

        o_ref[k * _L:(k + 1) * _L, :] = (
            tok + qq[:, :d] * rr[:, :d] + qq[:, d:] * rr[:, d:])


def kernel(entity, attribute, value_binned, time, W_entity, W_attribute, W_value_binned):
    B, S = entity.shape
    D = W_entity.shape[1]
    half = D // 2
    N = B * S
    G = N // _T

    ratio = math.log(10000.0) / half
    f = np.exp(-ratio * np.arange(half, dtype=np.float64))
    qa = (64.0 * np.arange(64, dtype=np.float64))[:, None] * f[None, :]
    ra = np.arange(64, dtype=np.float64)[:, None] * f[None, :]
    s1, c1 = np.sin(qa), np.cos(qa)
    s2, c2 = np.sin(ra), np.cos(ra)
    qtab = jnp.asarray(np.concatenate([s1, c1, c1, s1], axis=1),
                       dtype=jnp.bfloat16)
    rtab = jnp.asarray(np.concatenate([c2, c2, s2, -s2], axis=1),
                       dtype=jnp.bfloat16)

    w_all = jnp.concatenate(
        [W_entity, W_attribute, W_value_binned], axis=0).astype(jnp.bfloat16)

    def r3(x):
        return x.reshape(G, _TILES, _L)

    idx_spec = pl.BlockSpec((1, _TILES, _L), lambda i: (i, 0, 0))

    out = pl.pallas_call(
        _tc_body,
        grid=(G,),
        in_specs=[idx_spec, idx_spec, idx_spec, idx_spec,
                  pl.BlockSpec((w_all.shape[0], D), lambda i: (0, 0)),
                  pl.BlockSpec((64, 2 * D), lambda i: (0, 0)),
                  pl.BlockSpec((64, 2 * D), lambda i: (0, 0))],
        out_specs=pl.BlockSpec((_T, D), lambda i: (i, 0)),
        out_shape=jax.ShapeDtypeStruct((N, D), jnp.float32),
        compiler_params=pltpu.CompilerParams(
            dimension_semantics=("arbitrary",)),
    )(r3(entity), r3(attribute), r3(value_binned), r3(time), w_all, qtab, rtab)
    return out.reshape(B, S, D)
